# baseline (speedup 1.0000x reference)
import jax
import jax.numpy as jnp
from jax.experimental import pallas as pl
from jax.experimental.pallas import tpu as pltpu

_N = 8192
_H = 256
_B = 64


_QKV_BR = 1024


def _qkv_body(x_ref, wemb_ref, bemb_ref, wq_ref, bq_ref, wk_ref, bk_ref,
              wv_ref, bv_ref, q_ref, k_ref, v_ref):
    x = x_ref[...]
    h = jnp.maximum(
        jnp.dot(x, wemb_ref[...], preferred_element_type=jnp.float32)
        + bemb_ref[...], 0.0)
    scale = jnp.float32(1.4426950408889634) / jnp.sqrt(jnp.float32(_H))
    q = (jnp.dot(h, wq_ref[...], preferred_element_type=jnp.float32)
         + bq_ref[...]) * scale
    q_ref[...] = q.astype(jnp.bfloat16)
    k_ref[...] = (jnp.dot(h, wk_ref[...], preferred_element_type=jnp.float32)
                  + bk_ref[...]).astype(jnp.bfloat16)
    v_ref[...] = (jnp.dot(h, wv_ref[...], preferred_element_type=jnp.float32)
                  + bv_ref[...]).astype(jnp.bfloat16)


def _qkv(x, W_emb, b_emb, Wq, bq, Wk, bk, Wv, bv):
    row_spec = pl.BlockSpec((_QKV_BR, _H), lambda i: (i, 0))
    w_spec = pl.BlockSpec((_H, _H), lambda i: (0, 0))
    b_spec = pl.BlockSpec((1, _H), lambda i: (0, 0))
    return pl.pallas_call(
        _qkv_body,
        grid=(_N // _QKV_BR,),
        in_specs=[row_spec, w_spec, b_spec, w_spec, b_spec, w_spec, b_spec,
                  w_spec, b_spec],
        out_specs=[row_spec, row_spec, row_spec],
        out_shape=[jax.ShapeDtypeStruct((_N, _H), jnp.bfloat16)] * 3,
        compiler_params=pltpu.CompilerParams(
            dimension_semantics=("parallel",),
        ),
        name="qkv_proj",
    )(x, W_emb, b_emb.reshape(1, _H), Wq, bq.reshape(1, _H),
      Wk, bk.reshape(1, _H), Wv, bv.reshape(1, _H))


_ATTN_BQ = 1024
_ATTN_CK = 256


def _attn_body(q_ref, k_ref, v_ref, o_ref):
    q = q_ref[...]
    m = jnp.full((_ATTN_BQ, 1), -jnp.inf, jnp.float32)
    l = jnp.zeros((_ATTN_BQ, 1), jnp.float32)
    acc = jnp.zeros((_ATTN_BQ, _H), jnp.float32)
    for c in range(_N // _ATTN_CK):
        k_c = k_ref[c * _ATTN_CK:(c + 1) * _ATTN_CK, :]
        v_c = v_ref[c * _ATTN_CK:(c + 1) * _ATTN_CK, :]
        s = jax.lax.dot_general(q, k_c, (((1,), (1,)), ((), ())),
                                preferred_element_type=jnp.float32)
        m_new = jnp.maximum(m, jnp.max(s, axis=1, keepdims=True))
        p = jnp.exp2(s - m_new)
        corr = jnp.exp2(m - m_new)
        l = l * corr + jnp.sum(p, axis=1, keepdims=True)
        acc = acc * corr + jnp.dot(p.astype(jnp.bfloat16), v_c,
                                   preferred_element_type=jnp.float32)
        m = m_new
    o_ref[...] = acc / l


def _attn(q, k, v):
    q_spec = pl.BlockSpec((_ATTN_BQ, _H), lambda i: (i, 0))
    full_spec = pl.BlockSpec((_N, _H), lambda i: (0, 0))
    return pl.pallas_call(
        _attn_body,
        grid=(_N // _ATTN_BQ,),
        in_specs=[q_spec, full_spec, full_spec],
        out_specs=q_spec,
        out_shape=jax.ShapeDtypeStruct((_N, _H), jnp.float32),
        compiler_params=pltpu.CompilerParams(
            dimension_semantics=("parallel",),
            vmem_limit_bytes=56 * 1024 * 1024,
        ),
        name="flash_attn",
    )(q, k, v)


_POOL_BR = 512


def _pool_body(att_ref, segrow_ref, segcol_ref, seg_smem, out_ref,
               smax_ref, ssum_ref, cnt_ref):
    i = pl.program_id(0)
    nsteps = _N // _POOL_BR

    @pl.when(i == 0)
    def _():
        smax_ref[...] = jnp.full((_B, _H), -jnp.inf, jnp.float32)
        ssum_ref[...] = jnp.zeros((_B, _H), jnp.float32)
        cnt_ref[...] = jnp.zeros((_B, 1), jnp.float32)

    att = att_ref[...]
    segrow = segrow_ref[0]
    segcol = segcol_ref[0]

    ids = jax.lax.broadcasted_iota(jnp.int32, (_B, _POOL_BR), 0)
    mask = jnp.where(segrow == ids, 1.0, 0.0)
    ssum_ref[...] += jnp.dot(mask, att,
                             preferred_element_type=jnp.float32)
    cnt_ref[...] += jnp.sum(mask, axis=1, keepdims=True)

    lo = seg_smem[i * _POOL_BR]
    hi = seg_smem[i * _POOL_BR + _POOL_BR - 1]
    for b in range(_B):
        @pl.when((lo <= b) & (b <= hi))
        def _():
            masked = jnp.where(segcol == b, att, -jnp.inf)
            mx = jnp.max(masked, axis=0, keepdims=True)
            smax_ref[b:b + 1, :] = jnp.maximum(smax_ref[b:b + 1, :], mx)

    @pl.when(i == nsteps - 1)
    def _():
        cnt = cnt_ref[...]
        nonempty = cnt > 0.0
        mx = jnp.where(nonempty, smax_ref[...], 0.0)
        mean = jnp.where(nonempty,
                         ssum_ref[...] / jnp.maximum(cnt, 1.0), 0.0)
        out_ref[...] = jnp.concatenate([mx, mean], axis=1)


def _pool(att, seg):
    nsteps = _N // _POOL_BR
    segrow = seg.reshape(nsteps, 1, _POOL_BR)
    segcol = seg.reshape(nsteps, _POOL_BR, 1)
    return pl.pallas_call(
        _pool_body,
        grid=(nsteps,),
        in_specs=[
            pl.BlockSpec((_POOL_BR, _H), lambda i: (i, 0)),
            pl.BlockSpec((1, 1, _POOL_BR), lambda i: (i, 0, 0)),
            pl.BlockSpec((1, _POOL_BR, 1), lambda i: (i, 0, 0)),
            pl.BlockSpec(memory_space=pltpu.SMEM),
        ],
        out_specs=pl.BlockSpec((_B, 2 * _H), lambda i: (0, 0)),
        out_shape=jax.ShapeDtypeStruct((_B, 2 * _H), jnp.float32),
        scratch_shapes=[
            pltpu.VMEM((_B, _H), jnp.float32),
            pltpu.VMEM((_B, _H), jnp.float32),
            pltpu.VMEM((_B, 1), jnp.float32),
        ],
        compiler_params=pltpu.CompilerParams(
            dimension_semantics=("arbitrary",),
        ),
        name="segment_pool",
    )(att, segrow, segcol, seg)


def kernel(x, W_emb, b_emb, Wq, bq, Wk, bk, Wv, bv, batch):
    seg = batch.astype(jnp.int32)
    q, k, v = _qkv(x, W_emb, b_emb, Wq, bq, Wk, bk, Wv, bv)
    att = _attn(q, k, v)
    return _pool(att, seg)


# speedup vs baseline: 1.0453x; 1.0453x over previous
import jax
import jax.numpy as jnp
from jax.experimental import pallas as pl
from jax.experimental.pallas import tpu as pltpu

_N = 8192
_H = 256
_B = 64


_QKV_BR = 1024


def _qkv_body(x_ref, wemb_ref, bemb_ref, wq_ref, bq_ref, wk_ref, bk_ref,
              wv_ref, bv_ref, q_ref, k_ref, v_ref):
    x = x_ref[...]
    h = jnp.maximum(
        jnp.dot(x, wemb_ref[...], preferred_element_type=jnp.float32)
        + bemb_ref[...], 0.0)
    scale = jnp.float32(1.4426950408889634) / jnp.sqrt(jnp.float32(_H))
    q = (jnp.dot(h, wq_ref[...], preferred_element_type=jnp.float32)
         + bq_ref[...]) * scale
    q_ref[...] = q.astype(jnp.bfloat16)
    k_ref[...] = (jnp.dot(h, wk_ref[...], preferred_element_type=jnp.float32)
                  + bk_ref[...]).astype(jnp.bfloat16)
    v_ref[...] = (jnp.dot(h, wv_ref[...], preferred_element_type=jnp.float32)
                  + bv_ref[...]).astype(jnp.bfloat16)


def _qkv(x, W_emb, b_emb, Wq, bq, Wk, bk, Wv, bv):
    row_spec = pl.BlockSpec((_QKV_BR, _H), lambda i: (i, 0))
    w_spec = pl.BlockSpec((_H, _H), lambda i: (0, 0))
    b_spec = pl.BlockSpec((1, _H), lambda i: (0, 0))
    return pl.pallas_call(
        _qkv_body,
        grid=(_N // _QKV_BR,),
        in_specs=[row_spec, w_spec, b_spec, w_spec, b_spec, w_spec, b_spec,
                  w_spec, b_spec],
        out_specs=[row_spec, row_spec, row_spec],
        out_shape=[jax.ShapeDtypeStruct((_N, _H), jnp.bfloat16)] * 3,
        compiler_params=pltpu.CompilerParams(
            dimension_semantics=("parallel",),
        ),
        name="qkv_proj",
    )(x, W_emb, b_emb.reshape(1, _H), Wq, bq.reshape(1, _H),
      Wk, bk.reshape(1, _H), Wv, bv.reshape(1, _H))


_BQ = 1024
_CK = 512
_NSTEPS = _N // _BQ


def _attn_pool_body(q_ref, k_ref, v_ref, segrow_ref, segcol_ref, seg_smem,
                    out_ref, att_scr, smax_ref, ssum_ref, cnt_ref):
    i = pl.program_id(0)

    @pl.when(i == 0)
    def _():
        smax_ref[...] = jnp.full((_B, _H), -jnp.inf, jnp.float32)
        ssum_ref[...] = jnp.zeros((_B, _H), jnp.float32)
        cnt_ref[...] = jnp.zeros((_B, 1), jnp.float32)

    q = q_ref[...]
    m = jnp.full((_BQ, 1), -jnp.inf, jnp.float32)
    l = jnp.zeros((_BQ, 1), jnp.float32)
    acc = jnp.zeros((_BQ, _H), jnp.float32)
    for c in range(_N // _CK):
        k_c = k_ref[c * _CK:(c + 1) * _CK, :]
        v_c = v_ref[c * _CK:(c + 1) * _CK, :]
        s = jax.lax.dot_general(q, k_c, (((1,), (1,)), ((), ())),
                                preferred_element_type=jnp.float32)
        m_new = jnp.maximum(m, jnp.max(s, axis=1, keepdims=True))
        p = jnp.exp2(s - m_new)
        corr = jnp.exp2(m - m_new)
        l = l * corr + jnp.sum(p, axis=1, keepdims=True)
        acc = acc * corr + jnp.dot(p.astype(jnp.bfloat16), v_c,
                                   preferred_element_type=jnp.float32)
        m = m_new
    att_scr[...] = acc / l

    att = att_scr[...]
    segrow = segrow_ref[0]
    segcol = segcol_ref[0]

    ids = jax.lax.broadcasted_iota(jnp.int32, (_B, _BQ), 0)
    mask = jnp.where(segrow == ids, 1.0, 0.0)
    ssum_ref[...] += jnp.dot(mask, att,
                             preferred_element_type=jnp.float32)
    cnt_ref[...] += jnp.sum(mask, axis=1, keepdims=True)

    lo = seg_smem[i * _BQ]
    hi = seg_smem[i * _BQ + _BQ - 1]
    for b in range(_B):
        @pl.when((lo <= b) & (b <= hi))
        def _():
            masked = jnp.where(segcol == b, att, -jnp.inf)
            mx = jnp.max(masked, axis=0, keepdims=True)
            smax_ref[b:b + 1, :] = jnp.maximum(smax_ref[b:b + 1, :], mx)

    @pl.when(i == _NSTEPS - 1)
    def _():
        cnt = cnt_ref[...]
        nonempty = cnt > 0.0
        mx = jnp.where(nonempty, smax_ref[...], 0.0)
        mean = jnp.where(nonempty,
                         ssum_ref[...] / jnp.maximum(cnt, 1.0), 0.0)
        out_ref[...] = jnp.concatenate([mx, mean], axis=1)


def _attn_pool(q, k, v, seg):
    segrow = seg.reshape(_NSTEPS, 1, _BQ)
    segcol = seg.reshape(_NSTEPS, _BQ, 1)
    return pl.pallas_call(
        _attn_pool_body,
        grid=(_NSTEPS,),
        in_specs=[
            pl.BlockSpec((_BQ, _H), lambda i: (i, 0)),
            pl.BlockSpec((_N, _H), lambda i: (0, 0)),
            pl.BlockSpec((_N, _H), lambda i: (0, 0)),
            pl.BlockSpec((1, 1, _BQ), lambda i: (i, 0, 0)),
            pl.BlockSpec((1, _BQ, 1), lambda i: (i, 0, 0)),
            pl.BlockSpec(memory_space=pltpu.SMEM),
        ],
        out_specs=pl.BlockSpec((_B, 2 * _H), lambda i: (0, 0)),
        out_shape=jax.ShapeDtypeStruct((_B, 2 * _H), jnp.float32),
        scratch_shapes=[
            pltpu.VMEM((_BQ, _H), jnp.float32),
            pltpu.VMEM((_B, _H), jnp.float32),
            pltpu.VMEM((_B, _H), jnp.float32),
            pltpu.VMEM((_B, 1), jnp.float32),
        ],
        compiler_params=pltpu.CompilerParams(
            dimension_semantics=("arbitrary",),
            vmem_limit_bytes=56 * 1024 * 1024,
        ),
        name="attn_pool",
    )(q, k, v, segrow, segcol, seg)


def kernel(x, W_emb, b_emb, Wq, bq, Wk, bk, Wv, bv, batch):
    seg = batch.astype(jnp.int32)
    q, k, v = _qkv(x, W_emb, b_emb, Wq, bq, Wk, bk, Wv, bv)
    return _attn_pool(q, k, v, seg)


# speedup vs baseline: 1.1518x; 1.1019x over previous
import jax
import jax.numpy as jnp
from jax.experimental import pallas as pl
from jax.experimental.pallas import tpu as pltpu

_N = 8192
_H = 256
_B = 64


_QKV_BR = 1024
_NKB = _N // _QKV_BR


def _qkv_body(x_ref, wemb_ref, bemb_ref, wq_ref, bq_ref, wk_ref, bk_ref,
              wv_ref, bv_ref, q_ref, k_ref, v_ref, kbm_ref):
    x = x_ref[...]
    h = jnp.maximum(
        jnp.dot(x, wemb_ref[...], preferred_element_type=jnp.float32)
        + bemb_ref[...], 0.0)
    scale = jnp.float32(1.4426950408889634) / jnp.sqrt(jnp.float32(_H))
    q = (jnp.dot(h, wq_ref[...], preferred_element_type=jnp.float32)
         + bq_ref[...]) * scale
    q_ref[...] = q.astype(jnp.bfloat16)
    k = (jnp.dot(h, wk_ref[...], preferred_element_type=jnp.float32)
         + bk_ref[...])
    k_ref[...] = k.astype(jnp.bfloat16)
    v_ref[...] = (jnp.dot(h, wv_ref[...], preferred_element_type=jnp.float32)
                  + bv_ref[...]).astype(jnp.bfloat16)
    kn2 = jnp.sum(k * k, axis=1, keepdims=True)
    kbm_ref[...] = jnp.max(kn2, keepdims=True)[None]


def _qkv(x, W_emb, b_emb, Wq, bq, Wk, bk, Wv, bv):
    row_spec = pl.BlockSpec((_QKV_BR, _H), lambda i: (i, 0))
    w_spec = pl.BlockSpec((_H, _H), lambda i: (0, 0))
    b_spec = pl.BlockSpec((1, _H), lambda i: (0, 0))
    return pl.pallas_call(
        _qkv_body,
        grid=(_NKB,),
        in_specs=[row_spec, w_spec, b_spec, w_spec, b_spec, w_spec, b_spec,
                  w_spec, b_spec],
        out_specs=[row_spec, row_spec, row_spec,
                   pl.BlockSpec((1, 1, 1), lambda i: (i, 0, 0))],
        out_shape=[jax.ShapeDtypeStruct((_N, _H), jnp.bfloat16)] * 3
        + [jax.ShapeDtypeStruct((_NKB, 1, 1), jnp.float32)],
        compiler_params=pltpu.CompilerParams(
            dimension_semantics=("parallel",),
        ),
        name="qkv_proj",
    )(x, W_emb, b_emb.reshape(1, _H), Wq, bq.reshape(1, _H),
      Wk, bk.reshape(1, _H), Wv, bv.reshape(1, _H))


_BQ = 1024
_CK = 512
_NSTEPS = _N // _BQ


def _attn_pool_body(q_ref, k_ref, v_ref, segrow_ref, segcol_ref, seg_smem,
                    kbm_smem, out_ref, att_scr, smax_ref, ssum_ref, cnt_ref):
    i = pl.program_id(0)

    @pl.when(i == 0)
    def _():
        smax_ref[...] = jnp.full((_B, _H), -jnp.inf, jnp.float32)
        ssum_ref[...] = jnp.zeros((_B, _H), jnp.float32)
        cnt_ref[...] = jnp.zeros((_B, 1), jnp.float32)

    kmax2 = kbm_smem[0]
    for j in range(1, _NKB):
        kmax2 = jnp.maximum(kmax2, kbm_smem[j])
    q = q_ref[...]
    qf = q.astype(jnp.float32)
    qn2 = jnp.sum(qf * qf, axis=1, keepdims=True)
    bound = jnp.sqrt(qn2 * kmax2)
    l = jnp.zeros((_BQ, 1), jnp.float32)
    acc = jnp.zeros((_BQ, _H), jnp.float32)
    for c in range(_N // _CK):
        k_c = k_ref[c * _CK:(c + 1) * _CK, :]
        v_c = v_ref[c * _CK:(c + 1) * _CK, :]
        s = jax.lax.dot_general(q, k_c, (((1,), (1,)), ((), ())),
                                preferred_element_type=jnp.float32)
        p = jnp.exp2(s - bound)
        l = l + jnp.sum(p, axis=1, keepdims=True)
        acc = acc + jnp.dot(p.astype(jnp.bfloat16), v_c,
                            preferred_element_type=jnp.float32)
    att_scr[...] = acc / l

    att = att_scr[...]
    segrow = segrow_ref[0]
    segcol = segcol_ref[0]

    ids = jax.lax.broadcasted_iota(jnp.int32, (_B, _BQ), 0)
    mask = jnp.where(segrow == ids, 1.0, 0.0)
    ssum_ref[...] += jnp.dot(mask, att,
                             preferred_element_type=jnp.float32)
    cnt_ref[...] += jnp.sum(mask, axis=1, keepdims=True)

    lo = seg_smem[i * _BQ]
    hi = seg_smem[i * _BQ + _BQ - 1]
    for b in range(_B):
        @pl.when((lo <= b) & (b <= hi))
        def _():
            masked = jnp.where(segcol == b, att, -jnp.inf)
            mx = jnp.max(masked, axis=0, keepdims=True)
            smax_ref[b:b + 1, :] = jnp.maximum(smax_ref[b:b + 1, :], mx)

    @pl.when(i == _NSTEPS - 1)
    def _():
        cnt = cnt_ref[...]
        nonempty = cnt > 0.0
        mx = jnp.where(nonempty, smax_ref[...], 0.0)
        mean = jnp.where(nonempty,
                         ssum_ref[...] / jnp.maximum(cnt, 1.0), 0.0)
        out_ref[...] = jnp.concatenate([mx, mean], axis=1)


def _attn_pool(q, k, v, seg, kbm):
    segrow = seg.reshape(_NSTEPS, 1, _BQ)
    segcol = seg.reshape(_NSTEPS, _BQ, 1)
    return pl.pallas_call(
        _attn_pool_body,
        grid=(_NSTEPS,),
        in_specs=[
            pl.BlockSpec((_BQ, _H), lambda i: (i, 0)),
            pl.BlockSpec((_N, _H), lambda i: (0, 0)),
            pl.BlockSpec((_N, _H), lambda i: (0, 0)),
            pl.BlockSpec((1, 1, _BQ), lambda i: (i, 0, 0)),
            pl.BlockSpec((1, _BQ, 1), lambda i: (i, 0, 0)),
            pl.BlockSpec(memory_space=pltpu.SMEM),
            pl.BlockSpec(memory_space=pltpu.SMEM),
        ],
        out_specs=pl.BlockSpec((_B, 2 * _H), lambda i: (0, 0)),
        out_shape=jax.ShapeDtypeStruct((_B, 2 * _H), jnp.float32),
        scratch_shapes=[
            pltpu.VMEM((_BQ, _H), jnp.float32),
            pltpu.VMEM((_B, _H), jnp.float32),
            pltpu.VMEM((_B, _H), jnp.float32),
            pltpu.VMEM((_B, 1), jnp.float32),
        ],
        compiler_params=pltpu.CompilerParams(
            dimension_semantics=("arbitrary",),
            vmem_limit_bytes=56 * 1024 * 1024,
        ),
        name="attn_pool",
    )(q, k, v, segrow, segcol, seg, kbm)


def kernel(x, W_emb, b_emb, Wq, bq, Wk, bk, Wv, bv, batch):
    seg = batch.astype(jnp.int32)
    q, k, v, kbm = _qkv(x, W_emb, b_emb, Wq, bq, Wk, bk, Wv, bv)
    return _attn_pool(q, k, v, seg, kbm.reshape(_NKB))
